# Initial kernel scaffold; baseline (speedup 1.0000x reference)
import jax
import jax.numpy as jnp
from jax.experimental import pallas as pl


def kernel(x, edge_attr, params, edge_index):
    raise NotImplementedError("write your pallas kernel here")


# speedup vs baseline: 3.8585x; 3.8585x over previous
import functools

import jax
import jax.numpy as jnp
from jax import lax
from jax.experimental import pallas as pl
from jax.experimental.pallas import tpu as pltpu
from jax.experimental.pallas import tpu_sc as plsc

NE = 320000
NN = 10000
NNP = 10016
FW = 16
HID = 40

NC = 2
NS = 16
NW = NC * NS
EPW = NE // NW
CHUNK = 2000
NCHUNK = EPW // CHUNK
RPS = NNP // NS

BE = 6400


def _gather_body(h_hbm, dst_hbm, src_hbm, gd_hbm, gs_hbm, idx_v, rows_v, sem):
  c = lax.axis_index("c")
  s = lax.axis_index("s")
  wid = s * NC + c
  base = wid * EPW
  for k in range(NCHUNK):
    off = base + k * CHUNK
    pltpu.sync_copy(dst_hbm.at[pl.ds(off, CHUNK)], idx_v)
    pltpu.async_copy(h_hbm.at[idx_v], rows_v, sem).wait()
    pltpu.sync_copy(rows_v, gd_hbm.at[pl.ds(off, CHUNK)])
    pltpu.sync_copy(src_hbm.at[pl.ds(off, CHUNK)], idx_v)
    pltpu.async_copy(h_hbm.at[idx_v], rows_v, sem).wait()
    pltpu.sync_copy(rows_v, gs_hbm.at[pl.ds(off, CHUNK)])


@functools.cache
def _sc_gather():
  mesh = plsc.VectorSubcoreMesh(
      core_axis_name="c", subcore_axis_name="s", num_cores=NC,
      num_subcores=NS)
  return pl.kernel(
      _gather_body,
      out_type=(
          jax.ShapeDtypeStruct((NE, FW), jnp.float32),
          jax.ShapeDtypeStruct((NE, FW), jnp.float32),
      ),
      mesh=mesh,
      compiler_params=pltpu.CompilerParams(use_tc_tiling_on_sc=False),
      scratch_types=[
          pltpu.VMEM((CHUNK,), jnp.int32),
          pltpu.VMEM((CHUNK, FW), jnp.float32),
          pltpu.SemaphoreType.DMA,
      ],
  )


def _segsum_body(m_hbm, dst_hbm, agg_hbm, idx_v, m_v, agg_sp):
  c = lax.axis_index("c")
  s = lax.axis_index("s")
  wid = s * NC + c

  def zbody(i, carry):
    m_v[i, :] = jnp.zeros((FW,), jnp.float32)
    return carry

  lax.fori_loop(0, RPS, zbody, 0)
  pltpu.sync_copy(m_v.at[pl.ds(0, RPS)], agg_sp.at[pl.ds(s * RPS, RPS)])
  plsc.subcore_barrier()

  base = wid * EPW
  for k in range(NCHUNK):
    off = base + k * CHUNK
    pltpu.sync_copy(dst_hbm.at[pl.ds(off, CHUNK)], idx_v)
    pltpu.sync_copy(m_hbm.at[pl.ds(off, CHUNK)], m_v)
    pltpu.sync_copy(m_v, agg_sp.at[idx_v], add=True)
  plsc.subcore_barrier()

  pltpu.sync_copy(agg_sp.at[pl.ds(s * RPS, RPS)],
                  agg_hbm.at[c, pl.ds(s * RPS, RPS)])


@functools.cache
def _sc_segsum():
  mesh = plsc.VectorSubcoreMesh(
      core_axis_name="c", subcore_axis_name="s", num_cores=NC,
      num_subcores=NS)
  return pl.kernel(
      _segsum_body,
      out_type=jax.ShapeDtypeStruct((NC, NNP, FW), jnp.float32),
      mesh=mesh,
      compiler_params=pltpu.CompilerParams(use_tc_tiling_on_sc=False),
      scratch_types=[
          pltpu.VMEM((CHUNK,), jnp.int32),
          pltpu.VMEM((CHUNK, FW), jnp.float32),
          pltpu.VMEM_SHARED((NNP, FW), jnp.float32),
      ],
  )


def _dot(a, b):
  return jnp.dot(a, b, preferred_element_type=jnp.float32)


def _node_enc_body(x_ref, w1_ref, w2_ref, out_ref):
  z = jnp.maximum(_dot(x_ref[...], w1_ref[...]), 0.0)
  out_ref[...] = jnp.maximum(_dot(z, w2_ref[...]), 0.0)


def _edge_enc_body(a_ref, w1_ref, w2_ref, out_ref):
  z = jnp.maximum(_dot(a_ref[...], w1_ref[...]), 0.0)
  out_ref[...] = jnp.maximum(_dot(z, w2_ref[...]), 0.0)


def _rel_body(gd_ref, gs_ref, e_ref, w1_ref, b1_ref, w2_ref, b2_ref, w3_ref,
              b3_ref, out_ref):
  cat = jnp.concatenate([gd_ref[...], gs_ref[...], e_ref[...]], axis=1)
  z1 = jnp.maximum(_dot(cat, w1_ref[...]) + b1_ref[...], 0.0)
  z2 = jnp.maximum(_dot(z1, w2_ref[...]) + b2_ref[...], 0.0)
  out_ref[...] = _dot(z2, w3_ref[...]) + b3_ref[...]


def _obj_body(h_ref, agg_ref, w1_ref, b1_ref, w2_ref, b2_ref, w3_ref, b3_ref,
              out_ref):
  h = h_ref[...]
  agg = agg_ref[0] + agg_ref[1]
  cat = jnp.concatenate([h, agg], axis=1)
  z1 = jnp.maximum(_dot(cat, w1_ref[...]) + b1_ref[...], 0.0)
  z2 = jnp.maximum(_dot(z1, w2_ref[...]) + b2_ref[...], 0.0)
  hn = _dot(z2, w3_ref[...]) + b3_ref[...]
  out_ref[...] = 0.5 * (hn + h)


def _head_body(gs_ref, gd_ref, e_ref, w1_ref, b1_ref, w2_ref, b2_ref, w3_ref,
               b3_ref, out_ref):
  cat = jnp.concatenate([gs_ref[...], gd_ref[...], e_ref[...]], axis=1)
  z1 = jnp.maximum(_dot(cat, w1_ref[...]) + b1_ref[...], 0.0)
  z2 = jnp.maximum(_dot(z1, w2_ref[...]) + b2_ref[...], 0.0)
  logit = _dot(z2, w3_ref[...]) + b3_ref[...]
  out_ref[...] = jax.nn.sigmoid(logit[:, 0]).reshape(1, 1, BE)


def _full(shape):
  return pl.BlockSpec(shape, lambda i: tuple(0 for _ in shape))


def _eblk(w):
  return pl.BlockSpec((BE, w), lambda i: (i, 0))


_node_enc = pl.pallas_call(
    _node_enc_body,
    grid=(5,),
    in_specs=[pl.BlockSpec((2000, 128), lambda i: (i, 0)),
              _full((128, HID)), _full((HID, FW))],
    out_specs=pl.BlockSpec((2000, FW), lambda i: (i, 0)),
    out_shape=jax.ShapeDtypeStruct((NN, FW), jnp.float32),
)

_edge_enc = pl.pallas_call(
    _edge_enc_body,
    grid=(NE // BE,),
    in_specs=[_eblk(4), _full((4, HID)), _full((HID, FW))],
    out_specs=_eblk(FW),
    out_shape=jax.ShapeDtypeStruct((NE, FW), jnp.float32),
)

_rel = pl.pallas_call(
    _rel_body,
    grid=(NE // BE,),
    in_specs=[_eblk(FW), _eblk(FW), _eblk(FW),
              _full((3 * FW, HID)), _full((HID,)),
              _full((HID, HID)), _full((HID,)),
              _full((HID, FW)), _full((FW,))],
    out_specs=_eblk(FW),
    out_shape=jax.ShapeDtypeStruct((NE, FW), jnp.float32),
)

_obj = pl.pallas_call(
    _obj_body,
    grid=(1,),
    in_specs=[pl.BlockSpec((NNP, FW), lambda i: (0, 0)),
              pl.BlockSpec((NC, NNP, FW), lambda i: (0, 0, 0)),
              _full((2 * FW, HID)), _full((HID,)),
              _full((HID, HID)), _full((HID,)),
              _full((HID, FW)), _full((FW,))],
    out_specs=pl.BlockSpec((NNP, FW), lambda i: (0, 0)),
    out_shape=jax.ShapeDtypeStruct((NNP, FW), jnp.float32),
)

_head = pl.pallas_call(
    _head_body,
    grid=(NE // BE,),
    in_specs=[_eblk(FW), _eblk(FW), _eblk(FW),
              _full((3 * FW, HID)), _full((HID,)),
              _full((HID, HID)), _full((HID,)),
              _full((HID, 1)), _full((1,))],
    out_specs=pl.BlockSpec((1, 1, BE), lambda i: (i, 0, 0)),
    out_shape=jax.ShapeDtypeStruct((NE // BE, 1, BE), jnp.float32),
)


def _pad_rows16(w):
  return jnp.pad(w, ((0, FW - w.shape[0]), (0, 0)))


def _pad_cols16(w):
  return jnp.pad(w, ((0, 0), (0, FW - w.shape[1])))


def _pad_vec16(b):
  return jnp.pad(b, (0, FW - b.shape[0]))


def kernel(x, edge_attr, params, edge_index):
  src = edge_index[0]
  dst = edge_index[1]

  ne = params["node_enc"]
  ee = params["edge_enc"]
  h0 = _node_enc(x, ne[0]["W"], _pad_cols16(ne[1]["W"]))
  e = _edge_enc(edge_attr, ee[0]["W"], _pad_cols16(ee[1]["W"]))
  h = jnp.pad(h0, ((0, NNP - NN), (0, 0)))

  for layer in params["resin"]:
    rw = layer["relational"]
    ow = layer["object"]
    w1 = jnp.concatenate([
        _pad_rows16(rw[0]["W"][0:5]),
        _pad_rows16(rw[0]["W"][5:10]),
        _pad_rows16(rw[0]["W"][10:14]),
    ], axis=0)
    gd, gs = _sc_gather()(h, dst, src)
    m = _rel(gd, gs, e, w1, rw[0]["b"], rw[1]["W"], rw[1]["b"],
             _pad_cols16(rw[2]["W"]), _pad_vec16(rw[2]["b"]))
    agg = _sc_segsum()(m, dst)
    ow1 = jnp.concatenate([
        _pad_rows16(ow[0]["W"][0:5]),
        _pad_rows16(ow[0]["W"][5:9]),
    ], axis=0)
    h = _obj(h, agg, ow1, ow[0]["b"], ow[1]["W"], ow[1]["b"],
             _pad_cols16(ow[2]["W"]), _pad_vec16(ow[2]["b"]))

  fw = params["W"]
  wsum = fw[0]["W"][10:14] + fw[0]["W"][14:18] + fw[0]["W"][18:22] \
      + fw[0]["W"][22:26]
  fw1 = jnp.concatenate([
      _pad_rows16(fw[0]["W"][0:5]),
      _pad_rows16(fw[0]["W"][5:10]),
      _pad_rows16(wsum),
  ], axis=0)
  gd, gs = _sc_gather()(h, dst, src)
  out = _head(gs, gd, e, fw1, fw[0]["b"], fw[1]["W"], fw[1]["b"],
              fw[2]["W"], fw[2]["b"])
  return out.reshape(NE, 1)


# speedup vs baseline: 10.6107x; 2.7499x over previous
import functools

import jax
import jax.numpy as jnp
from jax import lax
from jax.experimental import pallas as pl
from jax.experimental.pallas import tpu as pltpu
from jax.experimental.pallas import tpu_sc as plsc

NE = 320000
NN = 10000
NNP = 10016
FW = 16
HID = 40

NC = 2
NS = 16
NW = NC * NS
EPW = NE // NW
CHUNK = 2000
NCHUNK = EPW // CHUNK
RPS = NNP // NS

NEP = NE // 8
BP = 800
PK = 8


def _gather_body(h_hbm, dst_hbm, src_hbm, gd_hbm, gs_hbm, idx_v, rows_v, sem):
  c = lax.axis_index("c")
  s = lax.axis_index("s")
  wid = s * NC + c
  base = wid * EPW
  for k in range(NCHUNK):
    off = base + k * CHUNK
    pltpu.sync_copy(dst_hbm.at[pl.ds(off, CHUNK)], idx_v)
    pltpu.async_copy(h_hbm.at[idx_v], rows_v, sem).wait()
    pltpu.sync_copy(rows_v, gd_hbm.at[pl.ds(off, CHUNK)])
    pltpu.sync_copy(src_hbm.at[pl.ds(off, CHUNK)], idx_v)
    pltpu.async_copy(h_hbm.at[idx_v], rows_v, sem).wait()
    pltpu.sync_copy(rows_v, gs_hbm.at[pl.ds(off, CHUNK)])


@functools.cache
def _sc_gather():
  mesh = plsc.VectorSubcoreMesh(
      core_axis_name="c", subcore_axis_name="s", num_cores=NC,
      num_subcores=NS)
  return pl.kernel(
      _gather_body,
      out_type=(
          jax.ShapeDtypeStruct((NE, FW), jnp.float32),
          jax.ShapeDtypeStruct((NE, FW), jnp.float32),
      ),
      mesh=mesh,
      compiler_params=pltpu.CompilerParams(use_tc_tiling_on_sc=False),
      scratch_types=[
          pltpu.VMEM((CHUNK,), jnp.int32),
          pltpu.VMEM((CHUNK, FW), jnp.float32),
          pltpu.SemaphoreType.DMA,
      ],
  )


def _segsum_body(m_hbm, dst_hbm, agg_hbm, idx_v, m_v, agg_sp):
  c = lax.axis_index("c")
  s = lax.axis_index("s")
  wid = s * NC + c

  def zbody(i, carry):
    m_v[i, :] = jnp.zeros((FW,), jnp.float32)
    return carry

  lax.fori_loop(0, RPS, zbody, 0)
  pltpu.sync_copy(m_v.at[pl.ds(0, RPS)], agg_sp.at[pl.ds(s * RPS, RPS)])
  plsc.subcore_barrier()

  base = wid * EPW
  for k in range(NCHUNK):
    off = base + k * CHUNK
    pltpu.sync_copy(dst_hbm.at[pl.ds(off, CHUNK)], idx_v)
    pltpu.sync_copy(m_hbm.at[pl.ds(off, CHUNK)], m_v)
    pltpu.sync_copy(m_v, agg_sp.at[idx_v], add=True)
  plsc.subcore_barrier()

  pltpu.sync_copy(agg_sp.at[pl.ds(s * RPS, RPS)],
                  agg_hbm.at[c, pl.ds(s * RPS, RPS)])


@functools.cache
def _sc_segsum():
  mesh = plsc.VectorSubcoreMesh(
      core_axis_name="c", subcore_axis_name="s", num_cores=NC,
      num_subcores=NS)
  return pl.kernel(
      _segsum_body,
      out_type=jax.ShapeDtypeStruct((NC, NNP, FW), jnp.float32),
      mesh=mesh,
      compiler_params=pltpu.CompilerParams(use_tc_tiling_on_sc=False),
      scratch_types=[
          pltpu.VMEM((CHUNK,), jnp.int32),
          pltpu.VMEM((CHUNK, FW), jnp.float32),
          pltpu.VMEM_SHARED((NNP, FW), jnp.float32),
      ],
  )


def _dot(a, b):
  return jnp.dot(a, b, preferred_element_type=jnp.float32)


def _node_enc_body(x_ref, w1_ref, w2_ref, out_ref):
  z = jnp.maximum(_dot(x_ref[...], w1_ref[...]), 0.0)
  out_ref[...] = jnp.maximum(_dot(z, w2_ref[...]), 0.0)


def _edge_enc_body(a_ref, w1_ref, w2_ref, out_ref):
  z = jnp.maximum(_dot(a_ref[...], w1_ref[...]), 0.0)
  out_ref[...] = jnp.maximum(_dot(z, w2_ref[...]), 0.0)


def _rel_body(gd_ref, gs_ref, e_ref, wd_ref, ws_ref, we_ref, b1_ref, w2_ref,
              b2_ref, w3_ref, b3_ref, out_ref):
  z1 = jnp.maximum(
      _dot(gd_ref[...], wd_ref[...]) + _dot(gs_ref[...], ws_ref[...])
      + _dot(e_ref[...], we_ref[...]) + b1_ref[...], 0.0)
  z2 = jnp.maximum(_dot(z1, w2_ref[...]) + b2_ref[...], 0.0)
  out_ref[...] = _dot(z2, w3_ref[...]) + b3_ref[...]


def _obj_body(h_ref, agg_ref, w1_ref, b1_ref, w2_ref, b2_ref, w3_ref, b3_ref,
              out_ref):
  h = h_ref[...]
  agg = agg_ref[0] + agg_ref[1]
  cat = jnp.concatenate([h, agg], axis=1)
  z1 = jnp.maximum(_dot(cat, w1_ref[...]) + b1_ref[...], 0.0)
  z2 = jnp.maximum(_dot(z1, w2_ref[...]) + b2_ref[...], 0.0)
  hn = _dot(z2, w3_ref[...]) + b3_ref[...]
  out_ref[...] = 0.5 * (hn + h)


def _head_body(gs_ref, gd_ref, e_ref, ws_ref, wd_ref, we_ref, b1_ref, w2_ref,
               b2_ref, w3_ref, b3_ref, out_ref):
  z1 = jnp.maximum(
      _dot(gs_ref[...], ws_ref[...]) + _dot(gd_ref[...], wd_ref[...])
      + _dot(e_ref[...], we_ref[...]) + b1_ref[...], 0.0)
  z2 = jnp.maximum(_dot(z1, w2_ref[...]) + b2_ref[...], 0.0)
  logit = _dot(z2, w3_ref[...]) + b3_ref[...]
  out_ref[...] = jax.nn.sigmoid(logit)


def _full(shape):
  return pl.BlockSpec(shape, lambda i: tuple(0 for _ in shape))


def _pblk(w):
  return pl.BlockSpec((BP, w), lambda i: (i, 0))


HID8 = PK * HID

_node_enc = pl.pallas_call(
    _node_enc_body,
    grid=(5,),
    in_specs=[pl.BlockSpec((2000, 128), lambda i: (i, 0)),
              _full((128, HID)), _full((HID, FW))],
    out_specs=pl.BlockSpec((2000, FW), lambda i: (i, 0)),
    out_shape=jax.ShapeDtypeStruct((NN, FW), jnp.float32),
)

_edge_enc = pl.pallas_call(
    _edge_enc_body,
    grid=(NEP // BP,),
    in_specs=[_pblk(PK * 4), _full((PK * 4, HID8)), _full((HID8, 128))],
    out_specs=_pblk(128),
    out_shape=jax.ShapeDtypeStruct((NEP, 128), jnp.float32),
)

_rel = pl.pallas_call(
    _rel_body,
    grid=(NEP // BP,),
    in_specs=[_pblk(128), _pblk(128), _pblk(128),
              _full((128, HID8)), _full((128, HID8)), _full((128, HID8)),
              _full((HID8,)),
              _full((HID8, HID8)), _full((HID8,)),
              _full((HID8, 128)), _full((128,))],
    out_specs=_pblk(128),
    out_shape=jax.ShapeDtypeStruct((NEP, 128), jnp.float32),
)

_obj = pl.pallas_call(
    _obj_body,
    grid=(1,),
    in_specs=[pl.BlockSpec((NNP, FW), lambda i: (0, 0)),
              pl.BlockSpec((NC, NNP, FW), lambda i: (0, 0, 0)),
              _full((2 * FW, HID)), _full((HID,)),
              _full((HID, HID)), _full((HID,)),
              _full((HID, FW)), _full((FW,))],
    out_specs=pl.BlockSpec((NNP, FW), lambda i: (0, 0)),
    out_shape=jax.ShapeDtypeStruct((NNP, FW), jnp.float32),
)

_head = pl.pallas_call(
    _head_body,
    grid=(NEP // BP,),
    in_specs=[_pblk(128), _pblk(128), _pblk(128),
              _full((128, HID8)), _full((128, HID8)), _full((128, HID8)),
              _full((HID8,)),
              _full((HID8, HID8)), _full((HID8,)),
              _full((HID8, PK)), _full((PK,))],
    out_specs=_pblk(PK),
    out_shape=jax.ShapeDtypeStruct((NEP, PK), jnp.float32),
)


def _pad_rows16(w):
  return jnp.pad(w, ((0, FW - w.shape[0]), (0, 0)))


def _pad_cols16(w):
  return jnp.pad(w, ((0, 0), (0, FW - w.shape[1])))


def _pad_vec16(b):
  return jnp.pad(b, (0, FW - b.shape[0]))


def _bdiag(w):
  return jax.scipy.linalg.block_diag(*([w] * PK))


def _btile(b):
  return jnp.tile(b, PK)


def kernel(x, edge_attr, params, edge_index):
  src = edge_index[0]
  dst = edge_index[1]

  ne = params["node_enc"]
  ee = params["edge_enc"]
  h0 = _node_enc(x, ne[0]["W"], _pad_cols16(ne[1]["W"]))
  e = _edge_enc(edge_attr.reshape(NEP, PK * 4),
                _bdiag(ee[0]["W"]), _bdiag(_pad_cols16(ee[1]["W"])))
  h = jnp.pad(h0, ((0, NNP - NN), (0, 0)))

  for layer in params["resin"]:
    rw = layer["relational"]
    ow = layer["object"]
    wd = _bdiag(_pad_rows16(rw[0]["W"][0:5]))
    ws = _bdiag(_pad_rows16(rw[0]["W"][5:10]))
    we = _bdiag(_pad_rows16(rw[0]["W"][10:14]))
    gd, gs = _sc_gather()(h, dst, src)
    m = _rel(gd.reshape(NEP, 128), gs.reshape(NEP, 128), e,
             wd, ws, we, _btile(rw[0]["b"]),
             _bdiag(rw[1]["W"]), _btile(rw[1]["b"]),
             _bdiag(_pad_cols16(rw[2]["W"])), _btile(_pad_vec16(rw[2]["b"])))
    agg = _sc_segsum()(m.reshape(NE, FW), dst)
    ow1 = jnp.concatenate([
        _pad_rows16(ow[0]["W"][0:5]),
        _pad_rows16(ow[0]["W"][5:9]),
    ], axis=0)
    h = _obj(h, agg, ow1, ow[0]["b"], ow[1]["W"], ow[1]["b"],
             _pad_cols16(ow[2]["W"]), _pad_vec16(ow[2]["b"]))

  fw = params["W"]
  wsum = fw[0]["W"][10:14] + fw[0]["W"][14:18] + fw[0]["W"][18:22] \
      + fw[0]["W"][22:26]
  gd, gs = _sc_gather()(h, dst, src)
  out = _head(gs.reshape(NEP, 128), gd.reshape(NEP, 128), e,
              _bdiag(_pad_rows16(fw[0]["W"][0:5])),
              _bdiag(_pad_rows16(fw[0]["W"][5:10])),
              _bdiag(_pad_rows16(wsum)),
              _btile(fw[0]["b"]),
              _bdiag(fw[1]["W"]), _btile(fw[1]["b"]),
              _bdiag(fw[2]["W"]), _btile(fw[2]["b"]))
  return out.reshape(NE, 1)


# speedup vs baseline: 11.2490x; 1.0602x over previous
import functools

import jax
import jax.numpy as jnp
from jax import lax
from jax.experimental import pallas as pl
from jax.experimental.pallas import tpu as pltpu
from jax.experimental.pallas import tpu_sc as plsc

NE = 320000
NN = 10000
NNP = 10016
FW = 16
HID = 40

NC = 2
NS = 16
NW = NC * NS
EPW = NE // NW
CHUNK = 2000
NCHUNK = EPW // CHUNK
RPS = NNP // NS

NEP = NE // 8
BP = 800
PK = 8


def _gather_body(h_hbm, ei_hbm, gd_hbm, gs_hbm, idx_v, rows_v, sem):
  c = lax.axis_index("c")
  s = lax.axis_index("s")
  wid = s * NC + c
  base = wid * EPW
  for k in range(NCHUNK):
    off = base + k * CHUNK
    pltpu.sync_copy(ei_hbm.at[1, pl.ds(off, CHUNK)], idx_v)
    pltpu.async_copy(h_hbm.at[idx_v], rows_v, sem).wait()
    pltpu.sync_copy(rows_v, gd_hbm.at[pl.ds(off, CHUNK)])
    pltpu.sync_copy(ei_hbm.at[0, pl.ds(off, CHUNK)], idx_v)
    pltpu.async_copy(h_hbm.at[idx_v], rows_v, sem).wait()
    pltpu.sync_copy(rows_v, gs_hbm.at[pl.ds(off, CHUNK)])


@functools.cache
def _sc_gather():
  mesh = plsc.VectorSubcoreMesh(
      core_axis_name="c", subcore_axis_name="s", num_cores=NC,
      num_subcores=NS)
  return pl.kernel(
      _gather_body,
      out_type=(
          jax.ShapeDtypeStruct((NE, FW), jnp.float32),
          jax.ShapeDtypeStruct((NE, FW), jnp.float32),
      ),
      mesh=mesh,
      compiler_params=pltpu.CompilerParams(use_tc_tiling_on_sc=False),
      scratch_types=[
          pltpu.VMEM((CHUNK,), jnp.int32),
          pltpu.VMEM((CHUNK, FW), jnp.float32),
          pltpu.SemaphoreType.DMA,
      ],
  )


def _segsum_body(m_hbm, ei_hbm, agg_hbm, idx_v, m_v, agg_sp):
  c = lax.axis_index("c")
  s = lax.axis_index("s")
  wid = s * NC + c

  def zbody(i, carry):
    m_v[i, :] = jnp.zeros((FW,), jnp.float32)
    return carry

  lax.fori_loop(0, RPS, zbody, 0)
  pltpu.sync_copy(m_v.at[pl.ds(0, RPS)], agg_sp.at[pl.ds(s * RPS, RPS)])
  plsc.subcore_barrier()

  base = wid * EPW
  for k in range(NCHUNK):
    off = base + k * CHUNK
    pltpu.sync_copy(ei_hbm.at[1, pl.ds(off, CHUNK)], idx_v)
    pltpu.sync_copy(m_hbm.at[pl.ds(off, CHUNK)], m_v)
    pltpu.sync_copy(m_v, agg_sp.at[idx_v], add=True)
  plsc.subcore_barrier()

  pltpu.sync_copy(agg_sp.at[pl.ds(s * RPS, RPS)],
                  agg_hbm.at[c, pl.ds(s * RPS, RPS)])


@functools.cache
def _sc_segsum():
  mesh = plsc.VectorSubcoreMesh(
      core_axis_name="c", subcore_axis_name="s", num_cores=NC,
      num_subcores=NS)
  return pl.kernel(
      _segsum_body,
      out_type=jax.ShapeDtypeStruct((NC, NNP, FW), jnp.float32),
      mesh=mesh,
      compiler_params=pltpu.CompilerParams(use_tc_tiling_on_sc=False),
      scratch_types=[
          pltpu.VMEM((CHUNK,), jnp.int32),
          pltpu.VMEM((CHUNK, FW), jnp.float32),
          pltpu.VMEM_SHARED((NNP, FW), jnp.float32),
      ],
  )


def _dot(a, b):
  return jnp.dot(a, b, preferred_element_type=jnp.float32)


def _node_enc_body(x_ref, w1_ref, w2_ref, out_ref):
  z = jnp.maximum(_dot(x_ref[...], w1_ref[...]), 0.0)
  out_ref[...] = jnp.maximum(_dot(z, w2_ref[...]), 0.0)


def _edge_enc_body(a_ref, w1_ref, w2_ref, out_ref):
  z = jnp.maximum(_dot(a_ref[...], w1_ref[...]), 0.0)
  out_ref[...] = jnp.maximum(_dot(z, w2_ref[...]), 0.0)


def _rel_body(gd_ref, gs_ref, e_ref, wd_ref, ws_ref, we_ref, b1_ref, w2_ref,
              b2_ref, w3_ref, b3_ref, out_ref):
  z1 = jnp.maximum(
      _dot(gd_ref[...], wd_ref[...]) + _dot(gs_ref[...], ws_ref[...])
      + _dot(e_ref[...], we_ref[...]) + b1_ref[...], 0.0)
  z2 = jnp.maximum(_dot(z1, w2_ref[...]) + b2_ref[...], 0.0)
  out_ref[...] = _dot(z2, w3_ref[...]) + b3_ref[...]


def _obj_body(h_ref, agg_ref, w1_ref, b1_ref, w2_ref, b2_ref, w3_ref, b3_ref,
              out_ref):
  h = h_ref[...]
  agg = agg_ref[0] + agg_ref[1]
  cat = jnp.concatenate([h, agg], axis=1)
  z1 = jnp.maximum(_dot(cat, w1_ref[...]) + b1_ref[...], 0.0)
  z2 = jnp.maximum(_dot(z1, w2_ref[...]) + b2_ref[...], 0.0)
  hn = _dot(z2, w3_ref[...]) + b3_ref[...]
  out_ref[...] = 0.5 * (hn + h)


def _head_body(gs_ref, gd_ref, e_ref, ws_ref, wd_ref, we_ref, b1_ref, w2_ref,
               b2_ref, w3_ref, b3_ref, out_ref):
  z1 = jnp.maximum(
      _dot(gs_ref[...], ws_ref[...]) + _dot(gd_ref[...], wd_ref[...])
      + _dot(e_ref[...], we_ref[...]) + b1_ref[...], 0.0)
  z2 = jnp.maximum(_dot(z1, w2_ref[...]) + b2_ref[...], 0.0)
  logit = _dot(z2, w3_ref[...]) + b3_ref[...]
  out_ref[...] = jax.nn.sigmoid(logit)


def _full(shape):
  return pl.BlockSpec(shape, lambda i: tuple(0 for _ in shape))


def _pblk(w):
  return pl.BlockSpec((BP, w), lambda i: (i, 0))


HID8 = PK * HID

_node_enc = pl.pallas_call(
    _node_enc_body,
    grid=(5,),
    in_specs=[pl.BlockSpec((2000, 128), lambda i: (i, 0)),
              _full((128, HID)), _full((HID, FW))],
    out_specs=pl.BlockSpec((2000, FW), lambda i: (i, 0)),
    out_shape=jax.ShapeDtypeStruct((NN, FW), jnp.float32),
)

_edge_enc = pl.pallas_call(
    _edge_enc_body,
    grid=(NEP // BP,),
    in_specs=[_pblk(PK * 4), _full((PK * 4, HID8)), _full((HID8, 128))],
    out_specs=_pblk(128),
    out_shape=jax.ShapeDtypeStruct((NEP, 128), jnp.float32),
)

_rel = pl.pallas_call(
    _rel_body,
    grid=(NEP // BP,),
    in_specs=[_pblk(128), _pblk(128), _pblk(128),
              _full((128, HID8)), _full((128, HID8)), _full((128, HID8)),
              _full((HID8,)),
              _full((HID8, HID8)), _full((HID8,)),
              _full((HID8, 128)), _full((128,))],
    out_specs=_pblk(128),
    out_shape=jax.ShapeDtypeStruct((NEP, 128), jnp.float32),
)

_obj = pl.pallas_call(
    _obj_body,
    grid=(1,),
    in_specs=[pl.BlockSpec((NNP, FW), lambda i: (0, 0)),
              pl.BlockSpec((NC, NNP, FW), lambda i: (0, 0, 0)),
              _full((2 * FW, HID)), _full((HID,)),
              _full((HID, HID)), _full((HID,)),
              _full((HID, FW)), _full((FW,))],
    out_specs=pl.BlockSpec((NNP, FW), lambda i: (0, 0)),
    out_shape=jax.ShapeDtypeStruct((NNP, FW), jnp.float32),
)

_head = pl.pallas_call(
    _head_body,
    grid=(NEP // BP,),
    in_specs=[_pblk(128), _pblk(128), _pblk(128),
              _full((128, HID8)), _full((128, HID8)), _full((128, HID8)),
              _full((HID8,)),
              _full((HID8, HID8)), _full((HID8,)),
              _full((HID8, PK)), _full((PK,))],
    out_specs=_pblk(PK),
    out_shape=jax.ShapeDtypeStruct((NEP, PK), jnp.float32),
)


def _pad_rows16(w):
  return jnp.pad(w, ((0, FW - w.shape[0]), (0, 0)))


def _pad_cols16(w):
  return jnp.pad(w, ((0, 0), (0, FW - w.shape[1])))


def _pad_vec16(b):
  return jnp.pad(b, (0, FW - b.shape[0]))


def _bdiag(w):
  return jax.scipy.linalg.block_diag(*([w] * PK))


def _btile(b):
  return jnp.tile(b, PK)


def kernel(x, edge_attr, params, edge_index):
  ne = params["node_enc"]
  ee = params["edge_enc"]
  h0 = _node_enc(x, ne[0]["W"], _pad_cols16(ne[1]["W"]))
  ea_p = (edge_attr.T.reshape(4, NEP, PK).transpose(1, 2, 0)
          .reshape(NEP, PK * 4))
  e = _edge_enc(ea_p, _bdiag(ee[0]["W"]), _bdiag(_pad_cols16(ee[1]["W"])))
  h = jnp.pad(h0, ((0, NNP - NN), (0, 0)))

  for layer in params["resin"]:
    rw = layer["relational"]
    ow = layer["object"]
    wd = _bdiag(_pad_rows16(rw[0]["W"][0:5]))
    ws = _bdiag(_pad_rows16(rw[0]["W"][5:10]))
    we = _bdiag(_pad_rows16(rw[0]["W"][10:14]))
    gd, gs = _sc_gather()(h, edge_index)
    m = _rel(gd.reshape(NEP, 128), gs.reshape(NEP, 128), e,
             wd, ws, we, _btile(rw[0]["b"]),
             _bdiag(rw[1]["W"]), _btile(rw[1]["b"]),
             _bdiag(_pad_cols16(rw[2]["W"])), _btile(_pad_vec16(rw[2]["b"])))
    agg = _sc_segsum()(m.reshape(NE, FW), edge_index)
    ow1 = jnp.concatenate([
        _pad_rows16(ow[0]["W"][0:5]),
        _pad_rows16(ow[0]["W"][5:9]),
    ], axis=0)
    h = _obj(h, agg, ow1, ow[0]["b"], ow[1]["W"], ow[1]["b"],
             _pad_cols16(ow[2]["W"]), _pad_vec16(ow[2]["b"]))

  fw = params["W"]
  wsum = fw[0]["W"][10:14] + fw[0]["W"][14:18] + fw[0]["W"][18:22] \
      + fw[0]["W"][22:26]
  gd, gs = _sc_gather()(h, edge_index)
  out = _head(gs.reshape(NEP, 128), gd.reshape(NEP, 128), e,
              _bdiag(_pad_rows16(fw[0]["W"][0:5])),
              _bdiag(_pad_rows16(fw[0]["W"][5:10])),
              _bdiag(_pad_rows16(wsum)),
              _btile(fw[0]["b"]),
              _bdiag(fw[1]["W"]), _btile(fw[1]["b"]),
              _bdiag(fw[2]["W"]), _btile(fw[2]["b"]))
  return out.reshape(NE, 1)


# speedup vs baseline: 11.6933x; 1.0395x over previous
import functools

import jax
import jax.numpy as jnp
from jax import lax
from jax.experimental import pallas as pl
from jax.experimental.pallas import tpu as pltpu
from jax.experimental.pallas import tpu_sc as plsc

NE = 320000
NN = 10000
NNP = 10016
FW = 16
HID = 40

NC = 2
NS = 16
NW = NC * NS
EPW = NE // NW
CHUNK = 2000
NCHUNK = EPW // CHUNK
RPS = NNP // NS

NEP = NE // 8
BP = 800
PK = 8


def _gather_body(h_hbm, ei_hbm, gd_hbm, gs_hbm, idxd_v, idxs_v, rows_a,
                 rows_b, sem_a, sem_b):
  c = lax.axis_index("c")
  s = lax.axis_index("s")
  wid = s * NC + c
  base = wid * EPW
  pltpu.sync_copy(ei_hbm.at[1, pl.ds(base, EPW)], idxd_v)
  pltpu.sync_copy(ei_hbm.at[0, pl.ds(base, EPW)], idxs_v)
  bufs = (rows_a, rows_b)
  sems = (sem_a, sem_b)

  def idx_slice(t):
    ref = idxd_v if t % 2 == 0 else idxs_v
    return ref.at[pl.ds((t // 2) * CHUNK, CHUNK)]

  def out_slice(t):
    ref = gd_hbm if t % 2 == 0 else gs_hbm
    return ref.at[pl.ds(base + (t // 2) * CHUNK, CHUNK)]

  ntask = 2 * NCHUNK
  copies = [None, None]
  copies[0] = pltpu.async_copy(h_hbm.at[idx_slice(0)], bufs[0], sems[0])
  for t in range(ntask):
    if t + 1 < ntask:
      copies[(t + 1) % 2] = pltpu.async_copy(
          h_hbm.at[idx_slice(t + 1)], bufs[(t + 1) % 2], sems[(t + 1) % 2])
    copies[t % 2].wait()
    pltpu.sync_copy(bufs[t % 2], out_slice(t))


@functools.cache
def _sc_gather():
  mesh = plsc.VectorSubcoreMesh(
      core_axis_name="c", subcore_axis_name="s", num_cores=NC,
      num_subcores=NS)
  return pl.kernel(
      _gather_body,
      out_type=(
          jax.ShapeDtypeStruct((NE, FW), jnp.float32),
          jax.ShapeDtypeStruct((NE, FW), jnp.float32),
      ),
      mesh=mesh,
      compiler_params=pltpu.CompilerParams(use_tc_tiling_on_sc=False),
      scratch_types=[
          pltpu.VMEM((EPW,), jnp.int32),
          pltpu.VMEM((EPW,), jnp.int32),
          pltpu.VMEM((CHUNK, FW), jnp.float32),
          pltpu.VMEM((CHUNK, FW), jnp.float32),
          pltpu.SemaphoreType.DMA,
          pltpu.SemaphoreType.DMA,
      ],
  )


def _segsum_body(m_hbm, ei_hbm, agg_hbm, idx_v, m_a, m_b, agg_sp, sem_a,
                 sem_b):
  c = lax.axis_index("c")
  s = lax.axis_index("s")
  wid = s * NC + c
  base = wid * EPW

  for k in range(NCHUNK):
    pltpu.sync_copy(ei_hbm.at[1, pl.ds(base + k * CHUNK, CHUNK)],
                    idx_v.at[k])

  def zbody(i, carry):
    m_a[i, :] = jnp.zeros((FW,), jnp.float32)
    return carry

  lax.fori_loop(0, RPS, zbody, 0)
  pltpu.sync_copy(m_a.at[pl.ds(0, RPS)], agg_sp.at[pl.ds(s * RPS, RPS)])
  plsc.subcore_barrier()

  bufs = (m_a, m_b)
  sems = (sem_a, sem_b)
  copies = [None, None]
  copies[0] = pltpu.async_copy(m_hbm.at[pl.ds(base, CHUNK)], bufs[0],
                               sems[0])
  for k in range(NCHUNK):
    if k + 1 < NCHUNK:
      copies[(k + 1) % 2] = pltpu.async_copy(
          m_hbm.at[pl.ds(base + (k + 1) * CHUNK, CHUNK)], bufs[(k + 1) % 2],
          sems[(k + 1) % 2])
    copies[k % 2].wait()
    pltpu.sync_copy(bufs[k % 2], agg_sp.at[idx_v.at[k]], add=True)
  plsc.subcore_barrier()

  pltpu.sync_copy(agg_sp.at[pl.ds(s * RPS, RPS)],
                  agg_hbm.at[c, pl.ds(s * RPS, RPS)])


@functools.cache
def _sc_segsum():
  mesh = plsc.VectorSubcoreMesh(
      core_axis_name="c", subcore_axis_name="s", num_cores=NC,
      num_subcores=NS)
  return pl.kernel(
      _segsum_body,
      out_type=jax.ShapeDtypeStruct((NC, NNP, FW), jnp.float32),
      mesh=mesh,
      compiler_params=pltpu.CompilerParams(use_tc_tiling_on_sc=False),
      scratch_types=[
          pltpu.VMEM((NCHUNK, CHUNK), jnp.int32),
          pltpu.VMEM((CHUNK, FW), jnp.float32),
          pltpu.VMEM((CHUNK, FW), jnp.float32),
          pltpu.VMEM_SHARED((NNP, FW), jnp.float32),
          pltpu.SemaphoreType.DMA,
          pltpu.SemaphoreType.DMA,
      ],
  )


def _dot(a, b):
  return jnp.dot(a, b, preferred_element_type=jnp.float32)


def _node_enc_body(x_ref, w1_ref, w2_ref, out_ref):
  z = jnp.maximum(_dot(x_ref[...], w1_ref[...]), 0.0)
  out_ref[...] = jnp.maximum(_dot(z, w2_ref[...]), 0.0)


def _edge_enc_body(a_ref, w1_ref, w2_ref, out_ref):
  z = jnp.maximum(_dot(a_ref[...], w1_ref[...]), 0.0)
  out_ref[...] = jnp.maximum(_dot(z, w2_ref[...]), 0.0)


def _rel_body(gd_ref, gs_ref, e_ref, wd_ref, ws_ref, we_ref, b1_ref, w2_ref,
              b2_ref, w3_ref, b3_ref, out_ref):
  z1 = jnp.maximum(
      _dot(gd_ref[...], wd_ref[...]) + _dot(gs_ref[...], ws_ref[...])
      + _dot(e_ref[...], we_ref[...]) + b1_ref[...], 0.0)
  z2 = jnp.maximum(_dot(z1, w2_ref[...]) + b2_ref[...], 0.0)
  out_ref[...] = _dot(z2, w3_ref[...]) + b3_ref[...]


def _obj_body(h_ref, agg_ref, w1_ref, b1_ref, w2_ref, b2_ref, w3_ref, b3_ref,
              out_ref):
  h = h_ref[...]
  agg = agg_ref[0] + agg_ref[1]
  cat = jnp.concatenate([h, agg], axis=1)
  z1 = jnp.maximum(_dot(cat, w1_ref[...]) + b1_ref[...], 0.0)
  z2 = jnp.maximum(_dot(z1, w2_ref[...]) + b2_ref[...], 0.0)
  hn = _dot(z2, w3_ref[...]) + b3_ref[...]
  out_ref[...] = 0.5 * (hn + h)


def _head_body(gs_ref, gd_ref, e_ref, ws_ref, wd_ref, we_ref, b1_ref, w2_ref,
               b2_ref, w3_ref, b3_ref, out_ref):
  z1 = jnp.maximum(
      _dot(gs_ref[...], ws_ref[...]) + _dot(gd_ref[...], wd_ref[...])
      + _dot(e_ref[...], we_ref[...]) + b1_ref[...], 0.0)
  z2 = jnp.maximum(_dot(z1, w2_ref[...]) + b2_ref[...], 0.0)
  logit = _dot(z2, w3_ref[...]) + b3_ref[...]
  out_ref[...] = jax.nn.sigmoid(logit)


def _full(shape):
  return pl.BlockSpec(shape, lambda i: tuple(0 for _ in shape))


def _pblk(w):
  return pl.BlockSpec((BP, w), lambda i: (i, 0))


HID8 = PK * HID

_node_enc = pl.pallas_call(
    _node_enc_body,
    grid=(5,),
    in_specs=[pl.BlockSpec((2000, 128), lambda i: (i, 0)),
              _full((128, HID)), _full((HID, FW))],
    out_specs=pl.BlockSpec((2000, FW), lambda i: (i, 0)),
    out_shape=jax.ShapeDtypeStruct((NN, FW), jnp.float32),
)

_edge_enc = pl.pallas_call(
    _edge_enc_body,
    grid=(NEP // BP,),
    in_specs=[_pblk(PK * 4), _full((PK * 4, HID8)), _full((HID8, 128))],
    out_specs=_pblk(128),
    out_shape=jax.ShapeDtypeStruct((NEP, 128), jnp.float32),
)

_rel = pl.pallas_call(
    _rel_body,
    grid=(NEP // BP,),
    in_specs=[_pblk(128), _pblk(128), _pblk(128),
              _full((128, HID8)), _full((128, HID8)), _full((128, HID8)),
              _full((HID8,)),
              _full((HID8, HID8)), _full((HID8,)),
              _full((HID8, 128)), _full((128,))],
    out_specs=_pblk(128),
    out_shape=jax.ShapeDtypeStruct((NEP, 128), jnp.float32),
)

_obj = pl.pallas_call(
    _obj_body,
    grid=(1,),
    in_specs=[pl.BlockSpec((NNP, FW), lambda i: (0, 0)),
              pl.BlockSpec((NC, NNP, FW), lambda i: (0, 0, 0)),
              _full((2 * FW, HID)), _full((HID,)),
              _full((HID, HID)), _full((HID,)),
              _full((HID, FW)), _full((FW,))],
    out_specs=pl.BlockSpec((NNP, FW), lambda i: (0, 0)),
    out_shape=jax.ShapeDtypeStruct((NNP, FW), jnp.float32),
)

_head = pl.pallas_call(
    _head_body,
    grid=(NEP // BP,),
    in_specs=[_pblk(128), _pblk(128), _pblk(128),
              _full((128, HID8)), _full((128, HID8)), _full((128, HID8)),
              _full((HID8,)),
              _full((HID8, HID8)), _full((HID8,)),
              _full((HID8, PK)), _full((PK,))],
    out_specs=_pblk(PK),
    out_shape=jax.ShapeDtypeStruct((NEP, PK), jnp.float32),
)


def _pad_rows16(w):
  return jnp.pad(w, ((0, FW - w.shape[0]), (0, 0)))


def _pad_cols16(w):
  return jnp.pad(w, ((0, 0), (0, FW - w.shape[1])))


def _pad_vec16(b):
  return jnp.pad(b, (0, FW - b.shape[0]))


def _bdiag(w):
  return jax.scipy.linalg.block_diag(*([w] * PK))


def _btile(b):
  return jnp.tile(b, PK)


def kernel(x, edge_attr, params, edge_index):
  ne = params["node_enc"]
  ee = params["edge_enc"]
  h0 = _node_enc(x, ne[0]["W"], _pad_cols16(ne[1]["W"]))
  ea_p = (edge_attr.T.reshape(4, NEP, PK).transpose(1, 2, 0)
          .reshape(NEP, PK * 4))
  e = _edge_enc(ea_p, _bdiag(ee[0]["W"]), _bdiag(_pad_cols16(ee[1]["W"])))
  h = jnp.pad(h0, ((0, NNP - NN), (0, 0)))

  for layer in params["resin"]:
    rw = layer["relational"]
    ow = layer["object"]
    wd = _bdiag(_pad_rows16(rw[0]["W"][0:5]))
    ws = _bdiag(_pad_rows16(rw[0]["W"][5:10]))
    we = _bdiag(_pad_rows16(rw[0]["W"][10:14]))
    gd, gs = _sc_gather()(h, edge_index)
    m = _rel(gd.reshape(NEP, 128), gs.reshape(NEP, 128), e,
             wd, ws, we, _btile(rw[0]["b"]),
             _bdiag(rw[1]["W"]), _btile(rw[1]["b"]),
             _bdiag(_pad_cols16(rw[2]["W"])), _btile(_pad_vec16(rw[2]["b"])))
    agg = _sc_segsum()(m.reshape(NE, FW), edge_index)
    ow1 = jnp.concatenate([
        _pad_rows16(ow[0]["W"][0:5]),
        _pad_rows16(ow[0]["W"][5:9]),
    ], axis=0)
    h = _obj(h, agg, ow1, ow[0]["b"], ow[1]["W"], ow[1]["b"],
             _pad_cols16(ow[2]["W"]), _pad_vec16(ow[2]["b"]))

  fw = params["W"]
  wsum = fw[0]["W"][10:14] + fw[0]["W"][14:18] + fw[0]["W"][18:22] \
      + fw[0]["W"][22:26]
  gd, gs = _sc_gather()(h, edge_index)
  out = _head(gs.reshape(NEP, 128), gd.reshape(NEP, 128), e,
              _bdiag(_pad_rows16(fw[0]["W"][0:5])),
              _bdiag(_pad_rows16(fw[0]["W"][5:10])),
              _bdiag(_pad_rows16(wsum)),
              _btile(fw[0]["b"]),
              _bdiag(fw[1]["W"]), _btile(fw[1]["b"]),
              _bdiag(fw[2]["W"]), _btile(fw[2]["b"]))
  return out.reshape(NE, 1)


# speedup vs baseline: 13.5592x; 1.1596x over previous
import functools

import jax
import jax.numpy as jnp
from jax import lax
from jax.experimental import pallas as pl
from jax.experimental.pallas import tpu as pltpu
from jax.experimental.pallas import tpu_sc as plsc

NE = 320000
NN = 10000
NNP = 10016
FW = 16
HID = 40

NC = 2
NS = 16
NW = NC * NS
EPW = NE // NW
CHUNK = 2000
NCHUNK = EPW // CHUNK
RPS = NNP // NS

NEP = NE // 8
BP = 800
PK = 8


def _gather_body(h_hbm, ei_hbm, gd_hbm, gs_hbm, idxd_v, idxs_v, rows_a,
                 rows_b, sem_a, sem_b):
  c = lax.axis_index("c")
  s = lax.axis_index("s")
  wid = s * NC + c
  base = wid * EPW
  pltpu.sync_copy(ei_hbm.at[1, pl.ds(base, EPW)], idxd_v)
  pltpu.sync_copy(ei_hbm.at[0, pl.ds(base, EPW)], idxs_v)
  bufs = (rows_a, rows_b)
  sems = (sem_a, sem_b)

  def idx_slice(t):
    ref = idxd_v if t % 2 == 0 else idxs_v
    return ref.at[pl.ds((t // 2) * CHUNK, CHUNK)]

  def out_slice(t):
    ref = gd_hbm if t % 2 == 0 else gs_hbm
    return ref.at[pl.ds(base + (t // 2) * CHUNK, CHUNK)]

  ntask = 2 * NCHUNK
  copies = [None, None]
  copies[0] = pltpu.async_copy(h_hbm.at[idx_slice(0)], bufs[0], sems[0])
  for t in range(ntask):
    if t + 1 < ntask:
      copies[(t + 1) % 2] = pltpu.async_copy(
          h_hbm.at[idx_slice(t + 1)], bufs[(t + 1) % 2], sems[(t + 1) % 2])
    copies[t % 2].wait()
    pltpu.sync_copy(bufs[t % 2], out_slice(t))


@functools.cache
def _sc_gather():
  mesh = plsc.VectorSubcoreMesh(
      core_axis_name="c", subcore_axis_name="s", num_cores=NC,
      num_subcores=NS)
  return pl.kernel(
      _gather_body,
      out_type=(
          jax.ShapeDtypeStruct((NE, FW), jnp.float32),
          jax.ShapeDtypeStruct((NE, FW), jnp.float32),
      ),
      mesh=mesh,
      compiler_params=pltpu.CompilerParams(use_tc_tiling_on_sc=False),
      scratch_types=[
          pltpu.VMEM((EPW,), jnp.int32),
          pltpu.VMEM((EPW,), jnp.int32),
          pltpu.VMEM((CHUNK, FW), jnp.float32),
          pltpu.VMEM((CHUNK, FW), jnp.float32),
          pltpu.SemaphoreType.DMA,
          pltpu.SemaphoreType.DMA,
      ],
  )


def _segsum_body(m_hbm, ei_hbm, agg_hbm, idx_v, m_a, m_b, agg_sp, sem_a,
                 sem_b):
  c = lax.axis_index("c")
  s = lax.axis_index("s")
  wid = s * NC + c
  base = wid * EPW

  for k in range(NCHUNK):
    pltpu.sync_copy(ei_hbm.at[1, pl.ds(base + k * CHUNK, CHUNK)],
                    idx_v.at[k])

  def zbody(i, carry):
    m_a[i, :] = jnp.zeros((FW,), jnp.float32)
    return carry

  lax.fori_loop(0, RPS, zbody, 0)
  pltpu.sync_copy(m_a.at[pl.ds(0, RPS)], agg_sp.at[pl.ds(s * RPS, RPS)])
  plsc.subcore_barrier()

  bufs = (m_a, m_b)
  sems = (sem_a, sem_b)
  copies = [None, None]
  copies[0] = pltpu.async_copy(m_hbm.at[pl.ds(base, CHUNK)], bufs[0],
                               sems[0])
  for k in range(NCHUNK):
    if k + 1 < NCHUNK:
      copies[(k + 1) % 2] = pltpu.async_copy(
          m_hbm.at[pl.ds(base + (k + 1) * CHUNK, CHUNK)], bufs[(k + 1) % 2],
          sems[(k + 1) % 2])
    copies[k % 2].wait()
    pltpu.sync_copy(bufs[k % 2], agg_sp.at[idx_v.at[k]], add=True)
  plsc.subcore_barrier()

  pltpu.sync_copy(agg_sp.at[pl.ds(s * RPS, RPS)],
                  agg_hbm.at[c, pl.ds(s * RPS, RPS)])


@functools.cache
def _sc_segsum():
  mesh = plsc.VectorSubcoreMesh(
      core_axis_name="c", subcore_axis_name="s", num_cores=NC,
      num_subcores=NS)
  return pl.kernel(
      _segsum_body,
      out_type=jax.ShapeDtypeStruct((NC, NNP, FW), jnp.float32),
      mesh=mesh,
      compiler_params=pltpu.CompilerParams(use_tc_tiling_on_sc=False),
      scratch_types=[
          pltpu.VMEM((NCHUNK, CHUNK), jnp.int32),
          pltpu.VMEM((CHUNK, FW), jnp.float32),
          pltpu.VMEM((CHUNK, FW), jnp.float32),
          pltpu.VMEM_SHARED((NNP, FW), jnp.float32),
          pltpu.SemaphoreType.DMA,
          pltpu.SemaphoreType.DMA,
      ],
  )


RPK_W = 20
RPK_B = 125
RPK_SUB = 25


def _repack_body(ea_hbm, eap_hbm, in_v, out_v):
  c = lax.axis_index("c")
  s = lax.axis_index("s")
  wid = s * NC + c

  @pl.when(wid < RPK_W)
  def _():
    lane = lax.broadcasted_iota(jnp.int32, (16,), 0)
    sub_row = jnp.where(lane >= 8, 1, 0).astype(jnp.int32)
    col_base = (lane - sub_row * 8) * 4
    for sub in range(RPK_B // RPK_SUB):
      blk0 = wid * RPK_B + sub * RPK_SUB
      pltpu.sync_copy(ea_hbm.at[pl.ds(blk0, RPK_SUB)], in_v)

      def body(b, carry):
        for f in range(4):
          colv = col_base + f
          for g in range(8):
            v = in_v[b, f, pl.ds(16 * g, 16)]
            rowv = sub_row + (b * 16 + 2 * g)
            plsc.store_scatter(out_v, [rowv, colv], v)
        return carry

      lax.fori_loop(0, RPK_SUB, body, 0)
      pltpu.sync_copy(out_v,
                      eap_hbm.at[pl.ds(blk0 * 16, RPK_SUB * 16)])


@functools.cache
def _sc_repack():
  mesh = plsc.VectorSubcoreMesh(
      core_axis_name="c", subcore_axis_name="s", num_cores=NC,
      num_subcores=NS)
  return pl.kernel(
      _repack_body,
      out_type=jax.ShapeDtypeStruct((NEP, PK * 4), jnp.float32),
      mesh=mesh,
      compiler_params=pltpu.CompilerParams(
          use_tc_tiling_on_sc=False, needs_layout_passes=False),
      scratch_types=[
          pltpu.VMEM((RPK_SUB, 4, 128), jnp.float32),
          pltpu.VMEM((RPK_SUB * 16, PK * 4), jnp.float32),
      ],
  )


def _dot(a, b):
  return jnp.dot(a, b, preferred_element_type=jnp.float32)


def _node_enc_body(x_ref, w1_ref, w2_ref, out_ref):
  z = jnp.maximum(_dot(x_ref[...], w1_ref[...]), 0.0)
  out_ref[...] = jnp.maximum(_dot(z, w2_ref[...]), 0.0)


def _edge_enc_body(a_ref, w1_ref, w2_ref, out_ref):
  z = jnp.maximum(_dot(a_ref[...], w1_ref[...]), 0.0)
  out_ref[...] = jnp.maximum(_dot(z, w2_ref[...]), 0.0)


def _rel_body(gd_ref, gs_ref, e_ref, wd_ref, ws_ref, we_ref, b1_ref, w2_ref,
              b2_ref, w3_ref, b3_ref, out_ref):
  z1 = jnp.maximum(
      _dot(gd_ref[...], wd_ref[...]) + _dot(gs_ref[...], ws_ref[...])
      + _dot(e_ref[...], we_ref[...]) + b1_ref[...], 0.0)
  z2 = jnp.maximum(_dot(z1, w2_ref[...]) + b2_ref[...], 0.0)
  out_ref[...] = _dot(z2, w3_ref[...]) + b3_ref[...]


def _obj_body(h_ref, agg_ref, w1_ref, b1_ref, w2_ref, b2_ref, w3_ref, b3_ref,
              out_ref):
  h = h_ref[...]
  agg = agg_ref[0] + agg_ref[1]
  cat = jnp.concatenate([h, agg], axis=1)
  z1 = jnp.maximum(_dot(cat, w1_ref[...]) + b1_ref[...], 0.0)
  z2 = jnp.maximum(_dot(z1, w2_ref[...]) + b2_ref[...], 0.0)
  hn = _dot(z2, w3_ref[...]) + b3_ref[...]
  out_ref[...] = 0.5 * (hn + h)


def _head_body(gs_ref, gd_ref, e_ref, ws_ref, wd_ref, we_ref, b1_ref, w2_ref,
               b2_ref, w3_ref, b3_ref, out_ref):
  z1 = jnp.maximum(
      _dot(gs_ref[...], ws_ref[...]) + _dot(gd_ref[...], wd_ref[...])
      + _dot(e_ref[...], we_ref[...]) + b1_ref[...], 0.0)
  z2 = jnp.maximum(_dot(z1, w2_ref[...]) + b2_ref[...], 0.0)
  logit = _dot(z2, w3_ref[...]) + b3_ref[...]
  out_ref[...] = jax.nn.sigmoid(logit)


def _full(shape):
  return pl.BlockSpec(shape, lambda i: tuple(0 for _ in shape))


def _pblk(w):
  return pl.BlockSpec((BP, w), lambda i: (i, 0))


HID8 = PK * HID

_node_enc = pl.pallas_call(
    _node_enc_body,
    grid=(5,),
    in_specs=[pl.BlockSpec((2000, 128), lambda i: (i, 0)),
              _full((128, HID)), _full((HID, FW))],
    out_specs=pl.BlockSpec((2000, FW), lambda i: (i, 0)),
    out_shape=jax.ShapeDtypeStruct((NN, FW), jnp.float32),
)

_edge_enc = pl.pallas_call(
    _edge_enc_body,
    grid=(NEP // BP,),
    in_specs=[_pblk(PK * 4), _full((PK * 4, HID8)), _full((HID8, 128))],
    out_specs=_pblk(128),
    out_shape=jax.ShapeDtypeStruct((NEP, 128), jnp.float32),
)

_rel = pl.pallas_call(
    _rel_body,
    grid=(NEP // BP,),
    in_specs=[_pblk(128), _pblk(128), _pblk(128),
              _full((128, HID8)), _full((128, HID8)), _full((128, HID8)),
              _full((HID8,)),
              _full((HID8, HID8)), _full((HID8,)),
              _full((HID8, 128)), _full((128,))],
    out_specs=_pblk(128),
    out_shape=jax.ShapeDtypeStruct((NEP, 128), jnp.float32),
)

_obj = pl.pallas_call(
    _obj_body,
    grid=(1,),
    in_specs=[pl.BlockSpec((NNP, FW), lambda i: (0, 0)),
              pl.BlockSpec((NC, NNP, FW), lambda i: (0, 0, 0)),
              _full((2 * FW, HID)), _full((HID,)),
              _full((HID, HID)), _full((HID,)),
              _full((HID, FW)), _full((FW,))],
    out_specs=pl.BlockSpec((NNP, FW), lambda i: (0, 0)),
    out_shape=jax.ShapeDtypeStruct((NNP, FW), jnp.float32),
)

_head = pl.pallas_call(
    _head_body,
    grid=(NEP // BP,),
    in_specs=[_pblk(128), _pblk(128), _pblk(128),
              _full((128, HID8)), _full((128, HID8)), _full((128, HID8)),
              _full((HID8,)),
              _full((HID8, HID8)), _full((HID8,)),
              _full((HID8, PK)), _full((PK,))],
    out_specs=_pblk(PK),
    out_shape=jax.ShapeDtypeStruct((NEP, PK), jnp.float32),
)


def _pad_rows16(w):
  return jnp.pad(w, ((0, FW - w.shape[0]), (0, 0)))


def _pad_cols16(w):
  return jnp.pad(w, ((0, 0), (0, FW - w.shape[1])))


def _pad_vec16(b):
  return jnp.pad(b, (0, FW - b.shape[0]))


def _bdiag(w):
  return jax.scipy.linalg.block_diag(*([w] * PK))


def _btile(b):
  return jnp.tile(b, PK)


def kernel(x, edge_attr, params, edge_index):
  ne = params["node_enc"]
  ee = params["edge_enc"]
  h0 = _node_enc(x, ne[0]["W"], _pad_cols16(ne[1]["W"]))
  ea_native = edge_attr.reshape(NE // 128, 128, 4).transpose(0, 2, 1)
  ea_p = _sc_repack()(ea_native)
  e = _edge_enc(ea_p, _bdiag(ee[0]["W"]), _bdiag(_pad_cols16(ee[1]["W"])))
  h = jnp.pad(h0, ((0, NNP - NN), (0, 0)))

  for layer in params["resin"]:
    rw = layer["relational"]
    ow = layer["object"]
    wd = _bdiag(_pad_rows16(rw[0]["W"][0:5]))
    ws = _bdiag(_pad_rows16(rw[0]["W"][5:10]))
    we = _bdiag(_pad_rows16(rw[0]["W"][10:14]))
    gd, gs = _sc_gather()(h, edge_index)
    m = _rel(gd.reshape(NEP, 128), gs.reshape(NEP, 128), e,
             wd, ws, we, _btile(rw[0]["b"]),
             _bdiag(rw[1]["W"]), _btile(rw[1]["b"]),
             _bdiag(_pad_cols16(rw[2]["W"])), _btile(_pad_vec16(rw[2]["b"])))
    agg = _sc_segsum()(m.reshape(NE, FW), edge_index)
    ow1 = jnp.concatenate([
        _pad_rows16(ow[0]["W"][0:5]),
        _pad_rows16(ow[0]["W"][5:9]),
    ], axis=0)
    h = _obj(h, agg, ow1, ow[0]["b"], ow[1]["W"], ow[1]["b"],
             _pad_cols16(ow[2]["W"]), _pad_vec16(ow[2]["b"]))

  fw = params["W"]
  wsum = fw[0]["W"][10:14] + fw[0]["W"][14:18] + fw[0]["W"][18:22] \
      + fw[0]["W"][22:26]
  gd, gs = _sc_gather()(h, edge_index)
  out = _head(gs.reshape(NEP, 128), gd.reshape(NEP, 128), e,
              _bdiag(_pad_rows16(fw[0]["W"][0:5])),
              _bdiag(_pad_rows16(fw[0]["W"][5:10])),
              _bdiag(_pad_rows16(wsum)),
              _btile(fw[0]["b"]),
              _bdiag(fw[1]["W"]), _btile(fw[1]["b"]),
              _bdiag(fw[2]["W"]), _btile(fw[2]["b"]))
  return out.reshape(NE, 1)


# speedup vs baseline: 13.6585x; 1.0073x over previous
import functools

import jax
import jax.numpy as jnp
from jax import lax
from jax.experimental import pallas as pl
from jax.experimental.pallas import tpu as pltpu
from jax.experimental.pallas import tpu_sc as plsc

NE = 320000
NN = 10000
NNP = 10016
FW = 16
HID = 40

NC = 2
NS = 16
NW = NC * NS
EPW = NE // NW
CHUNK = 2000
NCHUNK = EPW // CHUNK
RPS = NNP // NS

NEP = NE // 8
BP = 800
PK = 8


def _gather_body(h_hbm, ei_hbm, gd_hbm, gs_hbm, idxd_v, idxs_v, rows_a,
                 rows_b, sem_a, sem_b):
  c = lax.axis_index("c")
  s = lax.axis_index("s")
  wid = s * NC + c
  base = wid * EPW
  pltpu.sync_copy(ei_hbm.at[1, pl.ds(base, EPW)], idxd_v)
  pltpu.sync_copy(ei_hbm.at[0, pl.ds(base, EPW)], idxs_v)
  bufs = (rows_a, rows_b)
  sems = (sem_a, sem_b)

  def idx_slice(t):
    ref = idxd_v if t % 2 == 0 else idxs_v
    return ref.at[pl.ds((t // 2) * CHUNK, CHUNK)]

  def out_slice(t):
    ref = gd_hbm if t % 2 == 0 else gs_hbm
    return ref.at[pl.ds(base + (t // 2) * CHUNK, CHUNK)]

  ntask = 2 * NCHUNK
  copies = [None, None]
  copies[0] = pltpu.async_copy(h_hbm.at[idx_slice(0)], bufs[0], sems[0])
  for t in range(ntask):
    if t + 1 < ntask:
      copies[(t + 1) % 2] = pltpu.async_copy(
          h_hbm.at[idx_slice(t + 1)], bufs[(t + 1) % 2], sems[(t + 1) % 2])
    copies[t % 2].wait()
    pltpu.sync_copy(bufs[t % 2], out_slice(t))


@functools.cache
def _sc_gather():
  mesh = plsc.VectorSubcoreMesh(
      core_axis_name="c", subcore_axis_name="s", num_cores=NC,
      num_subcores=NS)
  return pl.kernel(
      _gather_body,
      out_type=(
          jax.ShapeDtypeStruct((NE, FW), jnp.float32),
          jax.ShapeDtypeStruct((NE, FW), jnp.float32),
      ),
      mesh=mesh,
      compiler_params=pltpu.CompilerParams(use_tc_tiling_on_sc=False),
      scratch_types=[
          pltpu.VMEM((EPW,), jnp.int32),
          pltpu.VMEM((EPW,), jnp.int32),
          pltpu.VMEM((CHUNK, FW), jnp.float32),
          pltpu.VMEM((CHUNK, FW), jnp.float32),
          pltpu.SemaphoreType.DMA,
          pltpu.SemaphoreType.DMA,
      ],
  )


def _segsum_body(m_hbm, ei_hbm, agg_hbm, idx_v, m_a, m_b, agg_sp, sem_a,
                 sem_b):
  c = lax.axis_index("c")
  s = lax.axis_index("s")
  wid = s * NC + c
  base = wid * EPW

  for k in range(NCHUNK):
    pltpu.sync_copy(ei_hbm.at[1, pl.ds(base + k * CHUNK, CHUNK)],
                    idx_v.at[k])

  def zbody(i, carry):
    m_a[i, :] = jnp.zeros((FW,), jnp.float32)
    return carry

  lax.fori_loop(0, RPS, zbody, 0)
  pltpu.sync_copy(m_a.at[pl.ds(0, RPS)], agg_sp.at[pl.ds(s * RPS, RPS)])
  plsc.subcore_barrier()

  bufs = (m_a, m_b)
  sems = (sem_a, sem_b)
  copies = [None, None]
  copies[0] = pltpu.async_copy(m_hbm.at[pl.ds(base, CHUNK)], bufs[0],
                               sems[0])
  for k in range(NCHUNK):
    if k + 1 < NCHUNK:
      copies[(k + 1) % 2] = pltpu.async_copy(
          m_hbm.at[pl.ds(base + (k + 1) * CHUNK, CHUNK)], bufs[(k + 1) % 2],
          sems[(k + 1) % 2])
    copies[k % 2].wait()
    pltpu.sync_copy(bufs[k % 2], agg_sp.at[idx_v.at[k]], add=True)
  plsc.subcore_barrier()

  pltpu.sync_copy(agg_sp.at[pl.ds(s * RPS, RPS)],
                  agg_hbm.at[c, pl.ds(s * RPS, RPS)])


@functools.cache
def _sc_segsum():
  mesh = plsc.VectorSubcoreMesh(
      core_axis_name="c", subcore_axis_name="s", num_cores=NC,
      num_subcores=NS)
  return pl.kernel(
      _segsum_body,
      out_type=jax.ShapeDtypeStruct((NC, NNP, FW), jnp.float32),
      mesh=mesh,
      compiler_params=pltpu.CompilerParams(use_tc_tiling_on_sc=False),
      scratch_types=[
          pltpu.VMEM((NCHUNK, CHUNK), jnp.int32),
          pltpu.VMEM((CHUNK, FW), jnp.float32),
          pltpu.VMEM((CHUNK, FW), jnp.float32),
          pltpu.VMEM_SHARED((NNP, FW), jnp.float32),
          pltpu.SemaphoreType.DMA,
          pltpu.SemaphoreType.DMA,
      ],
  )


RPK_W = 20
RPK_B = 125
RPK_SUB = 25


def _repack_body(ea_hbm, eap_hbm, in_v, out_v):
  c = lax.axis_index("c")
  s = lax.axis_index("s")
  wid = s * NC + c

  @pl.when(wid < RPK_W)
  def _():
    lane = lax.broadcasted_iota(jnp.int32, (16,), 0)
    sub_row = jnp.where(lane >= 8, 1, 0).astype(jnp.int32)
    col_base = (lane - sub_row * 8) * FW
    zero = jnp.zeros((16,), jnp.float32)

    def zrow(r, carry):
      for g in range(8):
        out_v[r, pl.ds(16 * g, 16)] = zero
      return carry

    lax.fori_loop(0, RPK_SUB * 16, zrow, 0)
    for sub in range(RPK_B // RPK_SUB):
      blk0 = wid * RPK_B + sub * RPK_SUB
      pltpu.sync_copy(ea_hbm.at[pl.ds(blk0, RPK_SUB)], in_v)

      def body(b, carry):
        for f in range(4):
          colv = col_base + f
          for g in range(8):
            v = in_v[b, f, pl.ds(16 * g, 16)]
            rowv = sub_row + (b * 16 + 2 * g)
            plsc.store_scatter(out_v, [rowv, colv], v)
        return carry

      lax.fori_loop(0, RPK_SUB, body, 0)
      pltpu.sync_copy(out_v, eap_hbm.at[pl.ds(blk0 * 16, RPK_SUB * 16)])


@functools.cache
def _sc_repack():
  mesh = plsc.VectorSubcoreMesh(
      core_axis_name="c", subcore_axis_name="s", num_cores=NC,
      num_subcores=NS)
  return pl.kernel(
      _repack_body,
      out_type=jax.ShapeDtypeStruct((NEP, 128), jnp.float32),
      mesh=mesh,
      compiler_params=pltpu.CompilerParams(
          use_tc_tiling_on_sc=False, needs_layout_passes=False),
      scratch_types=[
          pltpu.VMEM((RPK_SUB, 4, 128), jnp.float32),
          pltpu.VMEM((RPK_SUB * 16, 128), jnp.float32),
      ],
  )


def _dot(a, b):
  return jnp.dot(a, b, preferred_element_type=jnp.float32)


def _node_enc_body(x_ref, w1_ref, w2_ref, out_ref):
  z = jnp.maximum(_dot(x_ref[...], w1_ref[...]), 0.0)
  out_ref[...] = jnp.maximum(_dot(z, w2_ref[...]), 0.0)


def _bf(x):
  return x.astype(jnp.bfloat16)


def _edge_enc_body(a_ref, w1_ref, w2_ref, out_ref):
  z = jnp.maximum(_dot(a_ref[...], w1_ref[...]), 0.0)
  out_ref[...] = jnp.maximum(_dot(_bf(z), w2_ref[...]), 0.0)


def _rel_body(gd_ref, gs_ref, e_ref, wd_ref, ws_ref, we_ref, b1_ref, w2_ref,
              b2_ref, w3_ref, b3_ref, out_ref):
  z1 = jnp.maximum(
      _dot(gd_ref[...], wd_ref[...]) + _dot(gs_ref[...], ws_ref[...])
      + _dot(e_ref[...], we_ref[...]) + b1_ref[...], 0.0)
  z2 = jnp.maximum(_dot(_bf(z1), w2_ref[...]) + b2_ref[...], 0.0)
  out_ref[...] = _dot(z2, w3_ref[...]) + b3_ref[...]


def _obj_body(h_ref, agg_ref, w1_ref, b1_ref, w2_ref, b2_ref, w3_ref, b3_ref,
              out_ref):
  h = h_ref[...]
  agg = agg_ref[0] + agg_ref[1]
  cat = jnp.concatenate([h, agg], axis=1)
  z1 = jnp.maximum(_dot(cat, w1_ref[...]) + b1_ref[...], 0.0)
  z2 = jnp.maximum(_dot(z1, w2_ref[...]) + b2_ref[...], 0.0)
  hn = _dot(z2, w3_ref[...]) + b3_ref[...]
  out_ref[...] = 0.5 * (hn + h)


def _head_body(gs_ref, gd_ref, e_ref, ws_ref, wd_ref, we_ref, b1_ref, w2_ref,
               b2_ref, w3_ref, b3_ref, out_ref):
  z1 = jnp.maximum(
      _dot(gs_ref[...], ws_ref[...]) + _dot(gd_ref[...], wd_ref[...])
      + _dot(e_ref[...], we_ref[...]) + b1_ref[...], 0.0)
  z2 = jnp.maximum(_dot(_bf(z1), w2_ref[...]) + b2_ref[...], 0.0)
  logit = _dot(z2, w3_ref[...]) + b3_ref[...]
  out_ref[...] = jax.nn.sigmoid(logit)


def _full(shape):
  return pl.BlockSpec(shape, lambda i: tuple(0 for _ in shape))


def _pblk(w):
  return pl.BlockSpec((BP, w), lambda i: (i, 0))


HID8 = PK * HID

_node_enc = pl.pallas_call(
    _node_enc_body,
    grid=(5,),
    in_specs=[pl.BlockSpec((2000, 128), lambda i: (i, 0)),
              _full((128, HID)), _full((HID, FW))],
    out_specs=pl.BlockSpec((2000, FW), lambda i: (i, 0)),
    out_shape=jax.ShapeDtypeStruct((NN, FW), jnp.float32),
)

_edge_enc = pl.pallas_call(
    _edge_enc_body,
    grid=(NEP // BP,),
    in_specs=[_pblk(128), _full((128, HID8)), _full((HID8, 128))],
    out_specs=_pblk(128),
    out_shape=jax.ShapeDtypeStruct((NEP, 128), jnp.float32),
)

_rel = pl.pallas_call(
    _rel_body,
    grid=(NEP // BP,),
    in_specs=[_pblk(128), _pblk(128), _pblk(128),
              _full((128, HID8)), _full((128, HID8)), _full((128, HID8)),
              _full((HID8,)),
              _full((HID8, HID8)), _full((HID8,)),
              _full((HID8, 128)), _full((128,))],
    out_specs=_pblk(128),
    out_shape=jax.ShapeDtypeStruct((NEP, 128), jnp.float32),
)

_obj = pl.pallas_call(
    _obj_body,
    grid=(1,),
    in_specs=[pl.BlockSpec((NNP, FW), lambda i: (0, 0)),
              pl.BlockSpec((NC, NNP, FW), lambda i: (0, 0, 0)),
              _full((2 * FW, HID)), _full((HID,)),
              _full((HID, HID)), _full((HID,)),
              _full((HID, FW)), _full((FW,))],
    out_specs=pl.BlockSpec((NNP, FW), lambda i: (0, 0)),
    out_shape=jax.ShapeDtypeStruct((NNP, FW), jnp.float32),
)

_head = pl.pallas_call(
    _head_body,
    grid=(NEP // BP,),
    in_specs=[_pblk(128), _pblk(128), _pblk(128),
              _full((128, HID8)), _full((128, HID8)), _full((128, HID8)),
              _full((HID8,)),
              _full((HID8, HID8)), _full((HID8,)),
              _full((HID8, PK)), _full((PK,))],
    out_specs=_pblk(PK),
    out_shape=jax.ShapeDtypeStruct((NEP, PK), jnp.float32),
)


def _pad_rows16(w):
  return jnp.pad(w, ((0, FW - w.shape[0]), (0, 0)))


def _pad_cols16(w):
  return jnp.pad(w, ((0, 0), (0, FW - w.shape[1])))


def _pad_vec16(b):
  return jnp.pad(b, (0, FW - b.shape[0]))


def _bdiag(w):
  return jax.scipy.linalg.block_diag(*([w] * PK))


def _btile(b):
  return jnp.tile(b, PK)


def _bf16(w):
  return w.astype(jnp.bfloat16)


def _flat128(a):
  return a.reshape(a.size).reshape(NEP, 128)


def kernel(x, edge_attr, params, edge_index):
  ne = params["node_enc"]
  ee = params["edge_enc"]
  h0 = _node_enc(x, ne[0]["W"], _pad_cols16(ne[1]["W"]))
  ea_native = edge_attr.reshape(NE // 128, 128, 4).transpose(0, 2, 1)
  ea_p = _sc_repack()(ea_native)
  e = _edge_enc(ea_p, _bdiag(_pad_rows16(ee[0]["W"])),
                _bf16(_bdiag(_pad_cols16(ee[1]["W"]))))
  h = jnp.pad(h0, ((0, NNP - NN), (0, 0)))

  for layer in params["resin"]:
    rw = layer["relational"]
    ow = layer["object"]
    wd = _bdiag(_pad_rows16(rw[0]["W"][0:5]))
    ws = _bdiag(_pad_rows16(rw[0]["W"][5:10]))
    we = _bdiag(_pad_rows16(rw[0]["W"][10:14]))
    gd, gs = _sc_gather()(h, edge_index)
    m = _rel(_flat128(gd), _flat128(gs), e,
             wd, ws, we, _btile(rw[0]["b"]),
             _bf16(_bdiag(rw[1]["W"])), _btile(rw[1]["b"]),
             _bdiag(_pad_cols16(rw[2]["W"])),
             _btile(_pad_vec16(rw[2]["b"])))
    agg = _sc_segsum()(m.reshape(m.size).reshape(NE, FW), edge_index)
    ow1 = jnp.concatenate([
        _pad_rows16(ow[0]["W"][0:5]),
        _pad_rows16(ow[0]["W"][5:9]),
    ], axis=0)
    h = _obj(h, agg, ow1, ow[0]["b"], ow[1]["W"], ow[1]["b"],
             _pad_cols16(ow[2]["W"]), _pad_vec16(ow[2]["b"]))

  fw = params["W"]
  wsum = fw[0]["W"][10:14] + fw[0]["W"][14:18] + fw[0]["W"][18:22] \
      + fw[0]["W"][22:26]
  gd, gs = _sc_gather()(h, edge_index)
  out = _head(_flat128(gs), _flat128(gd), e,
              _bdiag(_pad_rows16(fw[0]["W"][0:5])),
              _bdiag(_pad_rows16(fw[0]["W"][5:10])),
              _bdiag(_pad_rows16(wsum)),
              _btile(fw[0]["b"]),
              _bf16(_bdiag(fw[1]["W"])), _btile(fw[1]["b"]),
              _bdiag(fw[2]["W"]), _btile(fw[2]["b"]))
  return out.reshape(NE, 1)


# speedup vs baseline: 13.8554x; 1.0144x over previous
import functools

import jax
import jax.numpy as jnp
from jax import lax
from jax.experimental import pallas as pl
from jax.experimental.pallas import tpu as pltpu
from jax.experimental.pallas import tpu_sc as plsc

NE = 320000
NN = 10000
NNP = 10016
FW = 16
HID = 40

NC = 2
NS = 16
NW = NC * NS
EPW = NE // NW
CHUNK = 2000
NCHUNK = EPW // CHUNK
RPS = NNP // NS

NEP = NE // 8
BP = 800
PK = 8


def _gather_body(h_hbm, ei_hbm, gd_hbm, gs_hbm, idxd_v, idxs_v, rows_a,
                 rows_b, sem_a, sem_b):
  c = lax.axis_index("c")
  s = lax.axis_index("s")
  wid = s * NC + c
  base = wid * EPW
  pltpu.sync_copy(ei_hbm.at[1, pl.ds(base, EPW)], idxd_v)
  pltpu.sync_copy(ei_hbm.at[0, pl.ds(base, EPW)], idxs_v)
  bufs = (rows_a, rows_b)
  sems = (sem_a, sem_b)

  def idx_slice(t):
    ref = idxd_v if t % 2 == 0 else idxs_v
    return ref.at[pl.ds((t // 2) * CHUNK, CHUNK)]

  def out_slice(t):
    ref = gd_hbm if t % 2 == 0 else gs_hbm
    return ref.at[pl.ds(base + (t // 2) * CHUNK, CHUNK)]

  ntask = 2 * NCHUNK
  copies = [None, None]
  copies[0] = pltpu.async_copy(h_hbm.at[idx_slice(0)], bufs[0], sems[0])
  for t in range(ntask):
    if t + 1 < ntask:
      copies[(t + 1) % 2] = pltpu.async_copy(
          h_hbm.at[idx_slice(t + 1)], bufs[(t + 1) % 2], sems[(t + 1) % 2])
    copies[t % 2].wait()
    pltpu.sync_copy(bufs[t % 2], out_slice(t))


@functools.cache
def _sc_gather():
  mesh = plsc.VectorSubcoreMesh(
      core_axis_name="c", subcore_axis_name="s", num_cores=NC,
      num_subcores=NS)
  return pl.kernel(
      _gather_body,
      out_type=(
          jax.ShapeDtypeStruct((NE, FW), jnp.float32),
          jax.ShapeDtypeStruct((NE, FW), jnp.float32),
      ),
      mesh=mesh,
      compiler_params=pltpu.CompilerParams(use_tc_tiling_on_sc=False),
      scratch_types=[
          pltpu.VMEM((EPW,), jnp.int32),
          pltpu.VMEM((EPW,), jnp.int32),
          pltpu.VMEM((CHUNK, FW), jnp.float32),
          pltpu.VMEM((CHUNK, FW), jnp.float32),
          pltpu.SemaphoreType.DMA,
          pltpu.SemaphoreType.DMA,
      ],
  )


def _segsum_body(m_hbm, ei_hbm, agg_hbm, idx_v, m_a, m_b, agg_sp, sem_a,
                 sem_b):
  c = lax.axis_index("c")
  s = lax.axis_index("s")
  wid = s * NC + c
  base = wid * EPW

  for k in range(NCHUNK):
    pltpu.sync_copy(ei_hbm.at[1, pl.ds(base + k * CHUNK, CHUNK)],
                    idx_v.at[k])

  def zbody(i, carry):
    m_a[i, :] = jnp.zeros((FW,), jnp.float32)
    return carry

  lax.fori_loop(0, RPS, zbody, 0)
  pltpu.sync_copy(m_a.at[pl.ds(0, RPS)], agg_sp.at[pl.ds(s * RPS, RPS)])
  plsc.subcore_barrier()

  bufs = (m_a, m_b)
  sems = (sem_a, sem_b)
  copies = [None, None]
  copies[0] = pltpu.async_copy(m_hbm.at[pl.ds(base, CHUNK)], bufs[0],
                               sems[0])
  for k in range(NCHUNK):
    if k + 1 < NCHUNK:
      copies[(k + 1) % 2] = pltpu.async_copy(
          m_hbm.at[pl.ds(base + (k + 1) * CHUNK, CHUNK)], bufs[(k + 1) % 2],
          sems[(k + 1) % 2])
    copies[k % 2].wait()
    pltpu.sync_copy(bufs[k % 2], agg_sp.at[idx_v.at[k]], add=True)
  plsc.subcore_barrier()

  pltpu.sync_copy(agg_sp.at[pl.ds(s * RPS, RPS)],
                  agg_hbm.at[c, pl.ds(s * RPS, RPS)])


@functools.cache
def _sc_segsum():
  mesh = plsc.VectorSubcoreMesh(
      core_axis_name="c", subcore_axis_name="s", num_cores=NC,
      num_subcores=NS)
  return pl.kernel(
      _segsum_body,
      out_type=jax.ShapeDtypeStruct((NC, NNP, FW), jnp.float32),
      mesh=mesh,
      compiler_params=pltpu.CompilerParams(use_tc_tiling_on_sc=False),
      scratch_types=[
          pltpu.VMEM((NCHUNK, CHUNK), jnp.int32),
          pltpu.VMEM((CHUNK, FW), jnp.float32),
          pltpu.VMEM((CHUNK, FW), jnp.float32),
          pltpu.VMEM_SHARED((NNP, FW), jnp.float32),
          pltpu.SemaphoreType.DMA,
          pltpu.SemaphoreType.DMA,
      ],
  )


RPK_W = 20
RPK_B = 125
RPK_SUB = 25


def _repack_body(ea_hbm, eap_hbm, in_v, out_v):
  c = lax.axis_index("c")
  s = lax.axis_index("s")
  wid = s * NC + c

  @pl.when(wid < RPK_W)
  def _():
    lane = lax.broadcasted_iota(jnp.int32, (16,), 0)
    sub_row = jnp.where(lane >= 8, 1, 0).astype(jnp.int32)
    col_base = (lane - sub_row * 8) * FW
    zero = jnp.zeros((16,), jnp.float32)

    def zrow(r, carry):
      for g in range(8):
        out_v[r, pl.ds(16 * g, 16)] = zero
      return carry

    lax.fori_loop(0, RPK_SUB * 16, zrow, 0)
    for sub in range(RPK_B // RPK_SUB):
      blk0 = wid * RPK_B + sub * RPK_SUB
      pltpu.sync_copy(ea_hbm.at[pl.ds(blk0, RPK_SUB)], in_v)

      def body(b, carry):
        for f in range(4):
          colv = col_base + f
          for g in range(8):
            v = in_v[b, f, pl.ds(16 * g, 16)]
            rowv = sub_row + (b * 16 + 2 * g)
            plsc.store_scatter(out_v, [rowv, colv], v)
        return carry

      lax.fori_loop(0, RPK_SUB, body, 0)
      pltpu.sync_copy(out_v, eap_hbm.at[pl.ds(blk0 * 16, RPK_SUB * 16)])


@functools.cache
def _sc_repack():
  mesh = plsc.VectorSubcoreMesh(
      core_axis_name="c", subcore_axis_name="s", num_cores=NC,
      num_subcores=NS)
  return pl.kernel(
      _repack_body,
      out_type=jax.ShapeDtypeStruct((NEP, 128), jnp.float32),
      mesh=mesh,
      compiler_params=pltpu.CompilerParams(
          use_tc_tiling_on_sc=False, needs_layout_passes=False),
      scratch_types=[
          pltpu.VMEM((RPK_SUB, 4, 128), jnp.float32),
          pltpu.VMEM((RPK_SUB * 16, 128), jnp.float32),
      ],
  )


def _dot(a, b):
  return jnp.dot(a, b, preferred_element_type=jnp.float32)


def _node_enc_body(x_ref, w1_ref, w2_ref, out_ref):
  z = jnp.maximum(_dot(x_ref[...], w1_ref[...]), 0.0)
  out_ref[...] = jnp.maximum(_dot(z, w2_ref[...]), 0.0)


def _bf(x):
  return x.astype(jnp.bfloat16)


def _edge_enc_body(a_ref, w1_ref, w2_ref, out_ref):
  z = jnp.maximum(_dot(a_ref[...], w1_ref[...]), 0.0)
  out_ref[...] = _bf(jnp.maximum(_dot(_bf(z), w2_ref[...]), 0.0))


def _rel_body(gd_ref, gs_ref, e_ref, wd_ref, ws_ref, we_ref, b1_ref, w2_ref,
              b2_ref, w3_ref, b3_ref, out_ref):
  z1 = jnp.maximum(
      _dot(gd_ref[...], wd_ref[...]) + _dot(gs_ref[...], ws_ref[...])
      + _dot(e_ref[...].astype(jnp.float32), we_ref[...]) + b1_ref[...], 0.0)
  z2 = jnp.maximum(_dot(_bf(z1), w2_ref[...]) + b2_ref[...], 0.0)
  out_ref[...] = _dot(z2, w3_ref[...]) + b3_ref[...]


def _obj_body(h_ref, agg_ref, w1_ref, b1_ref, w2_ref, b2_ref, w3_ref, b3_ref,
              out_ref):
  h = h_ref[...]
  agg = agg_ref[0] + agg_ref[1]
  cat = jnp.concatenate([h, agg], axis=1)
  z1 = jnp.maximum(_dot(cat, w1_ref[...]) + b1_ref[...], 0.0)
  z2 = jnp.maximum(_dot(z1, w2_ref[...]) + b2_ref[...], 0.0)
  hn = _dot(z2, w3_ref[...]) + b3_ref[...]
  out_ref[...] = 0.5 * (hn + h)


def _head_body(gs_ref, gd_ref, e_ref, ws_ref, wd_ref, we_ref, b1_ref, w2_ref,
               b2_ref, w3_ref, b3_ref, out_ref):
  z1 = jnp.maximum(
      _dot(gs_ref[...], ws_ref[...]) + _dot(gd_ref[...], wd_ref[...])
      + _dot(e_ref[...].astype(jnp.float32), we_ref[...]) + b1_ref[...], 0.0)
  z2 = jnp.maximum(_dot(_bf(z1), w2_ref[...]) + b2_ref[...], 0.0)
  logit = _dot(z2, w3_ref[...]) + b3_ref[...]
  out_ref[...] = jax.nn.sigmoid(logit)


def _full(shape):
  return pl.BlockSpec(shape, lambda i: tuple(0 for _ in shape))


def _pblk(w):
  return pl.BlockSpec((BP, w), lambda i: (i, 0))


HID8 = PK * HID

_node_enc = pl.pallas_call(
    _node_enc_body,
    grid=(5,),
    in_specs=[pl.BlockSpec((2000, 128), lambda i: (i, 0)),
              _full((128, HID)), _full((HID, FW))],
    out_specs=pl.BlockSpec((2000, FW), lambda i: (i, 0)),
    out_shape=jax.ShapeDtypeStruct((NN, FW), jnp.float32),
)

_edge_enc = pl.pallas_call(
    _edge_enc_body,
    grid=(NEP // BP,),
    in_specs=[_pblk(128), _full((128, HID8)), _full((HID8, 128))],
    out_specs=_pblk(128),
    out_shape=jax.ShapeDtypeStruct((NEP, 128), jnp.bfloat16),
)

_rel = pl.pallas_call(
    _rel_body,
    grid=(NEP // BP,),
    in_specs=[_pblk(128), _pblk(128), _pblk(128),
              _full((128, HID8)), _full((128, HID8)), _full((128, HID8)),
              _full((HID8,)),
              _full((HID8, HID8)), _full((HID8,)),
              _full((HID8, 128)), _full((128,))],
    out_specs=_pblk(128),
    out_shape=jax.ShapeDtypeStruct((NEP, 128), jnp.float32),
)

_obj = pl.pallas_call(
    _obj_body,
    grid=(1,),
    in_specs=[pl.BlockSpec((NNP, FW), lambda i: (0, 0)),
              pl.BlockSpec((NC, NNP, FW), lambda i: (0, 0, 0)),
              _full((2 * FW, HID)), _full((HID,)),
              _full((HID, HID)), _full((HID,)),
              _full((HID, FW)), _full((FW,))],
    out_specs=pl.BlockSpec((NNP, FW), lambda i: (0, 0)),
    out_shape=jax.ShapeDtypeStruct((NNP, FW), jnp.float32),
)

_head = pl.pallas_call(
    _head_body,
    grid=(NEP // BP,),
    in_specs=[_pblk(128), _pblk(128), _pblk(128),
              _full((128, HID8)), _full((128, HID8)), _full((128, HID8)),
              _full((HID8,)),
              _full((HID8, HID8)), _full((HID8,)),
              _full((HID8, PK)), _full((PK,))],
    out_specs=_pblk(PK),
    out_shape=jax.ShapeDtypeStruct((NEP, PK), jnp.float32),
)


def _pad_rows16(w):
  return jnp.pad(w, ((0, FW - w.shape[0]), (0, 0)))


def _pad_cols16(w):
  return jnp.pad(w, ((0, 0), (0, FW - w.shape[1])))


def _pad_vec16(b):
  return jnp.pad(b, (0, FW - b.shape[0]))


def _bdiag(w):
  return jax.scipy.linalg.block_diag(*([w] * PK))


def _btile(b):
  return jnp.tile(b, PK)


def _bf16(w):
  return w.astype(jnp.bfloat16)


def _flat128(a):
  return a.reshape(a.size).reshape(NEP, 128)


def kernel(x, edge_attr, params, edge_index):
  ne = params["node_enc"]
  ee = params["edge_enc"]
  h0 = _node_enc(x, ne[0]["W"], _pad_cols16(ne[1]["W"]))
  ea_native = edge_attr.reshape(NE // 128, 128, 4).transpose(0, 2, 1)
  ea_p = _sc_repack()(ea_native)
  e = _edge_enc(ea_p, _bdiag(_pad_rows16(ee[0]["W"])),
                _bf16(_bdiag(_pad_cols16(ee[1]["W"]))))
  h = jnp.pad(h0, ((0, NNP - NN), (0, 0)))

  for layer in params["resin"]:
    rw = layer["relational"]
    ow = layer["object"]
    wd = _bdiag(_pad_rows16(rw[0]["W"][0:5]))
    ws = _bdiag(_pad_rows16(rw[0]["W"][5:10]))
    we = _bdiag(_pad_rows16(rw[0]["W"][10:14]))
    gd, gs = _sc_gather()(h, edge_index)
    m = _rel(_flat128(gd), _flat128(gs), e,
             wd, ws, we, _btile(rw[0]["b"]),
             _bf16(_bdiag(rw[1]["W"])), _btile(rw[1]["b"]),
             _bdiag(_pad_cols16(rw[2]["W"])),
             _btile(_pad_vec16(rw[2]["b"])))
    agg = _sc_segsum()(m.reshape(m.size).reshape(NE, FW), edge_index)
    ow1 = jnp.concatenate([
        _pad_rows16(ow[0]["W"][0:5]),
        _pad_rows16(ow[0]["W"][5:9]),
    ], axis=0)
    h = _obj(h, agg, ow1, ow[0]["b"], ow[1]["W"], ow[1]["b"],
             _pad_cols16(ow[2]["W"]), _pad_vec16(ow[2]["b"]))

  fw = params["W"]
  wsum = fw[0]["W"][10:14] + fw[0]["W"][14:18] + fw[0]["W"][18:22] \
      + fw[0]["W"][22:26]
  gd, gs = _sc_gather()(h, edge_index)
  out = _head(_flat128(gs), _flat128(gd), e,
              _bdiag(_pad_rows16(fw[0]["W"][0:5])),
              _bdiag(_pad_rows16(fw[0]["W"][5:10])),
              _bdiag(_pad_rows16(wsum)),
              _btile(fw[0]["b"]),
              _bf16(_bdiag(fw[1]["W"])), _btile(fw[1]["b"]),
              _bdiag(fw[2]["W"]), _btile(fw[2]["b"]))
  return out.reshape(NE, 1)
